# baseline (speedup 1.0000x reference)
import functools
import math

import jax
import jax.numpy as jnp
from jax import lax
from jax.experimental import pallas as pl
from jax.experimental.pallas import tpu as pltpu
from jax.experimental.pallas import tpu_sc as plsc

HIDDEN = 1024
E = 3
L_SEQ = 2048
N_SEQ = 2
S = L_SEQ * N_SEQ
CAP = int(math.ceil(S / E))
CPAD = 1408
SLOTS = E * CPAD

BL = 512
NBLK = L_SEQ // BL

BC = 128

NC = 2
NS = 16
NW = NC * NS
TOK_W = S // NW
SUB = 64


def _routing_body(feat_ref, wg_ref, dst_ref, src_ref, gate_ref, aux_ref, acc_ref):
    n = pl.program_id(0)
    i = pl.program_id(1)
    first = jnp.logical_and(n == 0, i == 0)
    last = jnp.logical_and(n == N_SEQ - 1, i == NBLK - 1)

    @pl.when(first)
    def _():
        acc_ref[...] = jnp.zeros_like(acc_ref)

    x = feat_ref[...]
    logits = jnp.dot(x, wg_ref[...], preferred_element_type=jnp.float32)

    m = jnp.max(logits, axis=1, keepdims=True)
    p = jnp.exp(logits - m)
    denom = jnp.sum(p, axis=1, keepdims=True)
    gates = p / denom

    l0 = logits[:, 0:1]
    l1 = logits[:, 1:2]
    l2 = logits[:, 2:3]
    e = jnp.where(l1 > l0, 1, 0)
    e = jnp.where(l2 > jnp.maximum(l0, l1), 2, e)

    colid = lax.broadcasted_iota(jnp.int32, (BL, E), 1)
    mask = (colid == e).astype(jnp.float32)

    r = lax.broadcasted_iota(jnp.int32, (BL, BL), 0)
    c = lax.broadcasted_iota(jnp.int32, (BL, BL), 1)
    tri = (r >= c).astype(jnp.float32)
    prev = acc_ref[0:1, 0:E]
    cum = jnp.dot(tri, mask, preferred_element_type=jnp.float32) + prev
    loc = jnp.sum(cum * mask, axis=1, keepdims=True) - 1.0
    loc_i = loc.astype(jnp.int32)

    kept = loc_i < CAP
    dst = e * CPAD + jnp.where(kept, loc_i, CAP)
    gate = jnp.sum(gates * mask, axis=1, keepdims=True)
    gate_eff = jnp.where(kept, gate, 0.0)

    row = lax.broadcasted_iota(jnp.int32, (BL, 1), 0)
    src = (i * BL + row) * N_SEQ + n

    dst_ref[...] = dst
    src_ref[...] = src
    gate_ref[...] = gate_eff

    acc_ref[0:1, 0:E] = prev + jnp.sum(mask, axis=0, keepdims=True)
    acc_ref[1:2, 0:E] = acc_ref[1:2, 0:E] + jnp.sum(gates, axis=0, keepdims=True)

    aux = jnp.sum(acc_ref[0:1, 0:E] * acc_ref[1:2, 0:E], axis=1, keepdims=True)
    aux_ref[...] = aux * (E / (S * S))

    del last


def _routing(feat2, wg):
    return pl.pallas_call(
        _routing_body,
        grid=(N_SEQ, NBLK),
        in_specs=[
            pl.BlockSpec((BL, HIDDEN), lambda n, i: (i, n)),
            pl.BlockSpec((HIDDEN, E), lambda n, i: (0, 0)),
        ],
        out_specs=[
            pl.BlockSpec((BL, 1), lambda n, i: (n * NBLK + i, 0)),
            pl.BlockSpec((BL, 1), lambda n, i: (n * NBLK + i, 0)),
            pl.BlockSpec((BL, 1), lambda n, i: (n * NBLK + i, 0)),
            pl.BlockSpec((1, 1), lambda n, i: (0, 0)),
        ],
        out_shape=[
            jax.ShapeDtypeStruct((S, 1), jnp.int32),
            jax.ShapeDtypeStruct((S, 1), jnp.int32),
            jax.ShapeDtypeStruct((S, 1), jnp.float32),
            jax.ShapeDtypeStruct((1, 1), jnp.float32),
        ],
        scratch_shapes=[pltpu.VMEM((8, 128), jnp.float32)],
    )(feat2, wg)


_MESH = plsc.VectorSubcoreMesh(
    core_axis_name="c", subcore_axis_name="s", num_cores=NC, num_subcores=NS
)


@functools.partial(
    pl.kernel,
    out_type=jax.ShapeDtypeStruct((SLOTS,), jnp.float32),
    mesh=_MESH,
    compiler_params=pltpu.CompilerParams(needs_layout_passes=False),
    scratch_types=[
        pltpu.VMEM((S,), jnp.int32),
        pltpu.VMEM((S,), jnp.float32),
        pltpu.VMEM((SLOTS,), jnp.float32),
    ],
)
def _slotgate_k(dst_hbm, gate_hbm, out_hbm, dst_v, gate_v, sg_v):
    cid = lax.axis_index("c")
    sid = lax.axis_index("s")

    @pl.when(jnp.logical_and(cid == 0, sid == 0))
    def _():
        pltpu.sync_copy(dst_hbm, dst_v)
        pltpu.sync_copy(gate_hbm, gate_v)

        def init(j, carry):
            sg_v[pl.ds(j * 16, 16)] = jnp.zeros((16,), jnp.float32)
            return carry

        lax.fori_loop(0, SLOTS // 16, init, 0)

        def scat(j, carry):
            idx = dst_v[pl.ds(j * 16, 16)]
            val = gate_v[pl.ds(j * 16, 16)]
            plsc.store_scatter(sg_v, [idx], val)
            return carry

        lax.fori_loop(0, S // 16, scat, 0)
        pltpu.sync_copy(sg_v, out_hbm)


def _make_permute(out_rows):
    @functools.partial(
        pl.kernel,
        out_type=jax.ShapeDtypeStruct((out_rows, HIDDEN), jnp.float32),
        mesh=_MESH,
        scratch_types=[
            pltpu.VMEM((SUB,), jnp.int32),
            pltpu.VMEM((SUB,), jnp.int32),
            pltpu.VMEM((SUB, HIDDEN), jnp.float32),
            pltpu.SemaphoreType.DMA,
            pltpu.SemaphoreType.DMA,
        ],
    )
    def k(table_hbm, gidx_hbm, sidx_hbm, out_hbm, gi_v, si_v, rows_v, sem_g, sem_s):
        wid = lax.axis_index("s") * NC + lax.axis_index("c")
        for rnd in range(TOK_W // SUB):
            base = wid * TOK_W + rnd * SUB
            pltpu.sync_copy(gidx_hbm.at[pl.ds(base, SUB)], gi_v)
            pltpu.sync_copy(sidx_hbm.at[pl.ds(base, SUB)], si_v)
            pltpu.async_copy(table_hbm.at[gi_v], rows_v, sem_g).wait()
            pltpu.async_copy(rows_v, out_hbm.at[si_v], sem_s).wait()

    return k


_dispatch_k = _make_permute(SLOTS)
_combine_k = _make_permute(S)


def _mm_body(a_ref, w_ref, b_ref, g_ref, o_ref):
    acc = jnp.dot(a_ref[...], w_ref[0], preferred_element_type=jnp.float32)
    y = acc + b_ref[pl.ds(pl.program_id(0), 1), :]
    g = g_ref[...]
    o_ref[...] = jnp.where(g > 0.0, y * g, 0.0)


def _expert_mm(disp, We, be, sg):
    nblk = CPAD // BC
    return pl.pallas_call(
        _mm_body,
        grid=(E, nblk),
        in_specs=[
            pl.BlockSpec((BC, HIDDEN), lambda e, c: (e * nblk + c, 0)),
            pl.BlockSpec((1, HIDDEN, HIDDEN), lambda e, c: (e, 0, 0)),
            pl.BlockSpec((E, HIDDEN), lambda e, c: (0, 0)),
            pl.BlockSpec((BC, 1), lambda e, c: (e * nblk + c, 0)),
        ],
        out_specs=pl.BlockSpec((BC, HIDDEN), lambda e, c: (e * nblk + c, 0)),
        out_shape=jax.ShapeDtypeStruct((SLOTS, HIDDEN), jnp.float32),
    )(disp, We, be, sg)


def kernel(features, wg, We, be):
    feat2 = features.reshape(L_SEQ, N_SEQ * HIDDEN)
    dst, src, gate, aux = _routing(feat2, wg)
    dst1 = dst.reshape(S)
    src1 = src.reshape(S)
    gate1 = gate.reshape(S)

    sg = _slotgate_k(dst1, gate1)

    x = features.reshape(S, HIDDEN)
    disp = _dispatch_k(x, src1, dst1)
    eout = _expert_mm(disp, We, be, sg.reshape(SLOTS, 1))
    outp = _combine_k(eout, dst1, src1)

    out = outp.reshape(1, L_SEQ, N_SEQ, HIDDEN)
    return out, aux[0, 0]


# speedup vs baseline: 1.2155x; 1.2155x over previous
import functools
import math

import jax
import jax.numpy as jnp
from jax import lax
from jax.experimental import pallas as pl
from jax.experimental.pallas import tpu as pltpu
from jax.experimental.pallas import tpu_sc as plsc

HIDDEN = 1024
E = 3
L_SEQ = 2048
N_SEQ = 2
S = L_SEQ * N_SEQ
CAP = int(math.ceil(S / E))
CPAD = 1408
SLOTS = E * CPAD
HALF_SLOTS = SLOTS // 2

BL = 512
NBLK = L_SEQ // BL

BC = 128

NC = 2
NS = 16
NW = NC * NS
TOK_W = S // NW
SUB = 32
NRND = TOK_W // SUB


def _routing_body(feat_ref, wg_ref, dst_ref, src_ref, gate_ref, aux_ref,
                  xs_ref, acc_ref, tri_ref):
    n = pl.program_id(0)
    i = pl.program_id(1)
    first = jnp.logical_and(n == 0, i == 0)

    @pl.when(first)
    def _():
        acc_ref[...] = jnp.zeros_like(acc_ref)
        r = lax.broadcasted_iota(jnp.int32, (BL, BL), 0)
        c = lax.broadcasted_iota(jnp.int32, (BL, BL), 1)
        tri_ref[...] = (r >= c).astype(jnp.float32)

    x0 = feat_ref[0, :, 0, :]
    x1 = feat_ref[0, :, 1, :]
    x = jnp.where(n == 0, x0, x1)
    xs_ref[...] = x
    logits = jnp.dot(x, wg_ref[...], preferred_element_type=jnp.float32)

    m = jnp.max(logits, axis=1, keepdims=True)
    p = jnp.exp(logits - m)
    denom = jnp.sum(p, axis=1, keepdims=True)
    gates = p / denom

    l0 = logits[:, 0:1]
    l1 = logits[:, 1:2]
    l2 = logits[:, 2:3]
    e = jnp.where(l1 > l0, 1, 0)
    e = jnp.where(l2 > jnp.maximum(l0, l1), 2, e)

    colid = lax.broadcasted_iota(jnp.int32, (BL, E), 1)
    mask = (colid == e).astype(jnp.float32)

    prev = acc_ref[0:1, 0:E]
    cum = jnp.dot(tri_ref[...], mask, preferred_element_type=jnp.float32) + prev
    loc = jnp.sum(cum * mask, axis=1, keepdims=True) - 1.0
    loc_i = loc.astype(jnp.int32)

    kept = loc_i < CAP
    dst = e * CPAD + jnp.where(kept, loc_i, CAP)
    gate = jnp.sum(gates * mask, axis=1, keepdims=True)
    gate_eff = jnp.where(kept, gate, 0.0)

    row = lax.broadcasted_iota(jnp.int32, (BL, 1), 0)
    src = (i * BL + row) * N_SEQ + n

    dst_ref[...] = jnp.reshape(dst, (1, BL // 128, 128))
    src_ref[...] = jnp.reshape(src, (1, BL // 128, 128))
    gate_ref[...] = jnp.reshape(gate_eff, (1, BL // 128, 128))

    acc_ref[0:1, 0:E] = prev + jnp.sum(mask, axis=0, keepdims=True)
    acc_ref[1:2, 0:E] = acc_ref[1:2, 0:E] + jnp.sum(gates, axis=0, keepdims=True)

    aux = jnp.sum(acc_ref[0:1, 0:E] * acc_ref[1:2, 0:E], axis=1, keepdims=True)
    aux_ref[...] = aux * (E / (S * S))


def _routing(features, wg):
    qrows = BL // 128
    return pl.pallas_call(
        _routing_body,
        grid=(N_SEQ, NBLK),
        in_specs=[
            pl.BlockSpec((1, BL, N_SEQ, HIDDEN), lambda n, i: (0, i, 0, 0)),
            pl.BlockSpec((HIDDEN, E), lambda n, i: (0, 0)),
        ],
        out_specs=[
            pl.BlockSpec((1, qrows, 128), lambda n, i: (n * NBLK + i, 0, 0)),
            pl.BlockSpec((1, qrows, 128), lambda n, i: (n * NBLK + i, 0, 0)),
            pl.BlockSpec((1, qrows, 128), lambda n, i: (n * NBLK + i, 0, 0)),
            pl.BlockSpec((1, 1), lambda n, i: (0, 0)),
            pl.BlockSpec((BL, HIDDEN), lambda n, i: (n * NBLK + i, 0)),
        ],
        out_shape=[
            jax.ShapeDtypeStruct((N_SEQ * NBLK, qrows, 128), jnp.int32),
            jax.ShapeDtypeStruct((N_SEQ * NBLK, qrows, 128), jnp.int32),
            jax.ShapeDtypeStruct((N_SEQ * NBLK, qrows, 128), jnp.float32),
            jax.ShapeDtypeStruct((1, 1), jnp.float32),
            jax.ShapeDtypeStruct((S, HIDDEN), jnp.float32),
        ],
        scratch_shapes=[
            pltpu.VMEM((8, 128), jnp.float32),
            pltpu.VMEM((BL, BL), jnp.float32),
        ],
    )(features, wg)


_MESH = plsc.VectorSubcoreMesh(
    core_axis_name="c", subcore_axis_name="s", num_cores=NC, num_subcores=NS
)


def _ring_permute(table_hbm, out_hbm, gi_v, si_v, rows_v, gsems, ssems, wid,
                  gidx_hbm, sidx_hbm):
    base = wid * TOK_W
    for r in range(NRND):
        pltpu.sync_copy(gidx_hbm.at[pl.ds(base + r * SUB, SUB)], gi_v.at[r])
        pltpu.sync_copy(sidx_hbm.at[pl.ds(base + r * SUB, SUB)], si_v.at[r])
    g = [None] * NRND
    s = [None] * NRND
    g[0] = pltpu.async_copy(table_hbm.at[gi_v.at[0]], rows_v.at[0], gsems[0])
    g[1] = pltpu.async_copy(table_hbm.at[gi_v.at[1]], rows_v.at[1], gsems[1])
    for r in range(NRND):
        b = r % 2
        g[r].wait()
        s[r] = pltpu.async_copy(rows_v.at[b], out_hbm.at[si_v.at[r]], ssems[b])
        if r + 2 < NRND:
            s[r].wait()
            g[r + 2] = pltpu.async_copy(
                table_hbm.at[gi_v.at[r + 2]], rows_v.at[b], gsems[b]
            )
    s[NRND - 2].wait()
    s[NRND - 1].wait()


@functools.partial(
    pl.kernel,
    out_type=(
        jax.ShapeDtypeStruct((SLOTS, HIDDEN), jnp.float32),
        jax.ShapeDtypeStruct((SLOTS,), jnp.float32),
    ),
    mesh=_MESH,
    compiler_params=pltpu.CompilerParams(needs_layout_passes=False),
    scratch_types=[
        pltpu.VMEM((NRND, SUB), jnp.int32),
        pltpu.VMEM((NRND, SUB), jnp.int32),
        pltpu.VMEM((2, SUB, HIDDEN), jnp.float32),
        pltpu.VMEM((S,), jnp.int32),
        pltpu.VMEM((S,), jnp.float32),
        pltpu.VMEM((HALF_SLOTS,), jnp.float32),
        pltpu.SemaphoreType.DMA,
        pltpu.SemaphoreType.DMA,
        pltpu.SemaphoreType.DMA,
        pltpu.SemaphoreType.DMA,
    ],
)
def _dispatch_k(xs_hbm, dst_hbm, gate_hbm, disp_hbm, sg_hbm,
                gi_v, si_v, rows_v, dstall_v, gateall_v, sg_v,
                gsem0, gsem1, ssem0, ssem1):
    cid = lax.axis_index("c")
    sid = lax.axis_index("s")
    wid = sid * NC + cid

    @pl.when(sid == 0)
    def _():
        lo = cid * HALF_SLOTS
        pltpu.sync_copy(dst_hbm, dstall_v)
        pltpu.sync_copy(gate_hbm, gateall_v)

        def init(j, carry):
            sg_v[pl.ds(j * 16, 16)] = jnp.zeros((16,), jnp.float32)
            return carry

        lax.fori_loop(0, HALF_SLOTS // 16, init, 0)

        def scat(j, carry):
            idx = dstall_v[pl.ds(j * 16, 16)]
            val = gateall_v[pl.ds(j * 16, 16)]
            rel = idx - lo
            m = jnp.logical_and(rel >= 0, rel < HALF_SLOTS)
            rel = jnp.where(m, rel, 0)
            plsc.store_scatter(sg_v, [rel], val, mask=m)
            return carry

        lax.fori_loop(0, S // 16, scat, 0)
        pltpu.sync_copy(sg_v, sg_hbm.at[pl.ds(lo, HALF_SLOTS)])

    base = wid * TOK_W
    for r in range(NRND):
        pltpu.sync_copy(dst_hbm.at[pl.ds(base + r * SUB, SUB)], si_v.at[r])
    g = [None] * NRND
    s = [None] * NRND
    gsems = (gsem0, gsem1)
    ssems = (ssem0, ssem1)

    def gath(r, b, sem):
        return pltpu.async_copy(
            xs_hbm.at[pl.ds(base + r * SUB, SUB)], rows_v.at[b], sem
        )

    g[0] = gath(0, 0, gsems[0])
    g[1] = gath(1, 1, gsems[1])
    for r in range(NRND):
        b = r % 2
        g[r].wait()
        s[r] = pltpu.async_copy(rows_v.at[b], disp_hbm.at[si_v.at[r]], ssems[b])
        if r + 2 < NRND:
            s[r].wait()
            g[r + 2] = gath(r + 2, b, gsems[b])
    s[NRND - 2].wait()
    s[NRND - 1].wait()


@functools.partial(
    pl.kernel,
    out_type=jax.ShapeDtypeStruct((S, HIDDEN), jnp.float32),
    mesh=_MESH,
    compiler_params=pltpu.CompilerParams(needs_layout_passes=False),
    scratch_types=[
        pltpu.VMEM((NRND, SUB), jnp.int32),
        pltpu.VMEM((NRND, SUB), jnp.int32),
        pltpu.VMEM((2, SUB, HIDDEN), jnp.float32),
        pltpu.SemaphoreType.DMA,
        pltpu.SemaphoreType.DMA,
        pltpu.SemaphoreType.DMA,
        pltpu.SemaphoreType.DMA,
    ],
)
def _combine_k(eout_hbm, dst_hbm, src_hbm, out_hbm,
               gi_v, si_v, rows_v, gsem0, gsem1, ssem0, ssem1):
    cid = lax.axis_index("c")
    sid = lax.axis_index("s")
    wid = sid * NC + cid
    _ring_permute(eout_hbm, out_hbm, gi_v, si_v, rows_v,
                  (gsem0, gsem1), (ssem0, ssem1), wid, dst_hbm, src_hbm)


def _mm_body(a_ref, w_ref, b_ref, g_ref, o_ref):
    acc = jnp.dot(a_ref[...], w_ref[0], preferred_element_type=jnp.float32)
    y = acc + b_ref[pl.ds(pl.program_id(0), 1), :]
    g = g_ref[...]
    o_ref[...] = jnp.where(g > 0.0, y * g, 0.0)


def _expert_mm(disp, We, be, sg):
    nblk = CPAD // BC
    return pl.pallas_call(
        _mm_body,
        grid=(E, nblk),
        in_specs=[
            pl.BlockSpec((BC, HIDDEN), lambda e, c: (e * nblk + c, 0)),
            pl.BlockSpec((1, HIDDEN, HIDDEN), lambda e, c: (e, 0, 0)),
            pl.BlockSpec((E, HIDDEN), lambda e, c: (0, 0)),
            pl.BlockSpec((BC, 1), lambda e, c: (e * nblk + c, 0)),
        ],
        out_specs=pl.BlockSpec((BC, HIDDEN), lambda e, c: (e * nblk + c, 0)),
        out_shape=jax.ShapeDtypeStruct((SLOTS, HIDDEN), jnp.float32),
    )(disp, We, be, sg)


def kernel(features, wg, We, be):
    dst, src, gate, aux, xs = _routing(features, wg)
    dst1 = dst.reshape(S)
    src1 = src.reshape(S)
    gate1 = gate.reshape(S)

    disp, sg = _dispatch_k(xs, dst1, gate1)
    eout = _expert_mm(disp, We, be, sg.reshape(SLOTS, 1))
    outp = _combine_k(eout, dst1, src1)

    out = outp.reshape(1, L_SEQ, N_SEQ, HIDDEN)
    return out, aux[0, 0]


# speedup vs baseline: 1.2187x; 1.0027x over previous
import functools
import math

import jax
import jax.numpy as jnp
from jax import lax
from jax.experimental import pallas as pl
from jax.experimental.pallas import tpu as pltpu
from jax.experimental.pallas import tpu_sc as plsc

HIDDEN = 1024
E = 3
L_SEQ = 2048
N_SEQ = 2
S = L_SEQ * N_SEQ
CAP = int(math.ceil(S / E))
CPAD = 1408
SLOTS = E * CPAD
HALF_SLOTS = SLOTS // 2

BL = 512
NBLK = L_SEQ // BL

BC = 128

NC = 2
NS = 16
NW = NC * NS
TOK_W = S // NW
SUB = 32
NRND = TOK_W // SUB
NBUF = 3


def _routing_body(feat_ref, wg_ref, dst_ref, src_ref, gate_ref, aux_ref,
                  xs_ref, acc_ref, tri_ref):
    n = pl.program_id(0)
    i = pl.program_id(1)
    first = jnp.logical_and(n == 0, i == 0)

    @pl.when(first)
    def _():
        acc_ref[...] = jnp.zeros_like(acc_ref)
        r = lax.broadcasted_iota(jnp.int32, (BL, BL), 0)
        c = lax.broadcasted_iota(jnp.int32, (BL, BL), 1)
        tri_ref[...] = (r >= c).astype(jnp.float32)

    x0 = feat_ref[0, :, 0, :]
    x1 = feat_ref[0, :, 1, :]
    x = jnp.where(n == 0, x0, x1)
    xs_ref[...] = x
    logits = jnp.dot(x, wg_ref[...], preferred_element_type=jnp.float32)

    m = jnp.max(logits, axis=1, keepdims=True)
    p = jnp.exp(logits - m)
    denom = jnp.sum(p, axis=1, keepdims=True)
    gates = p / denom

    l0 = logits[:, 0:1]
    l1 = logits[:, 1:2]
    l2 = logits[:, 2:3]
    e = jnp.where(l1 > l0, 1, 0)
    e = jnp.where(l2 > jnp.maximum(l0, l1), 2, e)

    colid = lax.broadcasted_iota(jnp.int32, (BL, E), 1)
    mask = (colid == e).astype(jnp.float32)

    prev = acc_ref[0:1, 0:E]
    cum = jnp.dot(tri_ref[...], mask, preferred_element_type=jnp.float32) + prev
    loc = jnp.sum(cum * mask, axis=1, keepdims=True) - 1.0
    loc_i = loc.astype(jnp.int32)

    kept = loc_i < CAP
    dst = e * CPAD + jnp.where(kept, loc_i, CAP)
    gate = jnp.sum(gates * mask, axis=1, keepdims=True)
    gate_eff = jnp.where(kept, gate, 0.0)

    row = lax.broadcasted_iota(jnp.int32, (BL, 1), 0)
    src = (i * BL + row) * N_SEQ + n

    dst_ref[...] = jnp.reshape(dst, (1, BL // 128, 128))
    src_ref[...] = jnp.reshape(src, (1, BL // 128, 128))
    gate_ref[...] = jnp.reshape(gate_eff, (1, BL // 128, 128))

    acc_ref[0:1, 0:E] = prev + jnp.sum(mask, axis=0, keepdims=True)
    acc_ref[1:2, 0:E] = acc_ref[1:2, 0:E] + jnp.sum(gates, axis=0, keepdims=True)

    @pl.when(jnp.logical_and(n == N_SEQ - 1, i == NBLK - 1))
    def _():
        aux = jnp.sum(acc_ref[0:1, 0:E] * acc_ref[1:2, 0:E], axis=1,
                      keepdims=True)
        aux_ref[...] = aux * (E / (S * S))


def _routing(features, wg):
    qrows = BL // 128
    return pl.pallas_call(
        _routing_body,
        grid=(N_SEQ, NBLK),
        in_specs=[
            pl.BlockSpec((1, BL, N_SEQ, HIDDEN), lambda n, i: (0, i, 0, 0)),
            pl.BlockSpec((HIDDEN, E), lambda n, i: (0, 0)),
        ],
        out_specs=[
            pl.BlockSpec((1, qrows, 128), lambda n, i: (n * NBLK + i, 0, 0)),
            pl.BlockSpec((1, qrows, 128), lambda n, i: (n * NBLK + i, 0, 0)),
            pl.BlockSpec((1, qrows, 128), lambda n, i: (n * NBLK + i, 0, 0)),
            pl.BlockSpec((1, 1), lambda n, i: (0, 0)),
            pl.BlockSpec((BL, HIDDEN), lambda n, i: (n * NBLK + i, 0)),
        ],
        out_shape=[
            jax.ShapeDtypeStruct((N_SEQ * NBLK, qrows, 128), jnp.int32),
            jax.ShapeDtypeStruct((N_SEQ * NBLK, qrows, 128), jnp.int32),
            jax.ShapeDtypeStruct((N_SEQ * NBLK, qrows, 128), jnp.float32),
            jax.ShapeDtypeStruct((1, 1), jnp.float32),
            jax.ShapeDtypeStruct((S, HIDDEN), jnp.float32),
        ],
        scratch_shapes=[
            pltpu.VMEM((8, 128), jnp.float32),
            pltpu.VMEM((BL, BL), jnp.float32),
        ],
    )(features, wg)


_MESH = plsc.VectorSubcoreMesh(
    core_axis_name="c", subcore_axis_name="s", num_cores=NC, num_subcores=NS
)


def _run_ring(issue_gather, issue_scatter, gsems, ssems):
    g = [None] * NRND
    s = [None] * NRND
    for r in range(min(NBUF, NRND)):
        g[r] = issue_gather(r, r % NBUF, gsems[r % NBUF])
    for r in range(NRND):
        b = r % NBUF
        g[r].wait()
        s[r] = issue_scatter(r, b, ssems[b])
        if r + NBUF < NRND:
            s[r].wait()
            g[r + NBUF] = issue_gather(r + NBUF, b, gsems[b])
    for r in range(max(0, NRND - NBUF), NRND):
        s[r].wait()


@functools.partial(
    pl.kernel,
    out_type=(
        jax.ShapeDtypeStruct((SLOTS, HIDDEN), jnp.float32),
        jax.ShapeDtypeStruct((SLOTS,), jnp.float32),
    ),
    mesh=_MESH,
    compiler_params=pltpu.CompilerParams(needs_layout_passes=False),
    scratch_types=[
        pltpu.VMEM((NRND, SUB), jnp.int32),
        pltpu.VMEM((NRND, SUB), jnp.int32),
        pltpu.VMEM((NBUF, SUB, HIDDEN), jnp.float32),
        pltpu.VMEM((S,), jnp.int32),
        pltpu.VMEM((S,), jnp.float32),
        pltpu.VMEM((HALF_SLOTS,), jnp.float32),
        pltpu.SemaphoreType.DMA,
        pltpu.SemaphoreType.DMA,
        pltpu.SemaphoreType.DMA,
        pltpu.SemaphoreType.DMA,
        pltpu.SemaphoreType.DMA,
        pltpu.SemaphoreType.DMA,
    ],
)
def _dispatch_k(xs_hbm, dst_hbm, gate_hbm, disp_hbm, sg_hbm,
                gi_v, si_v, rows_v, dstall_v, gateall_v, sg_v,
                gsem0, gsem1, gsem2, ssem0, ssem1, ssem2):
    cid = lax.axis_index("c")
    sid = lax.axis_index("s")
    wid = sid * NC + cid

    @pl.when(sid == 0)
    def _():
        lo = cid * HALF_SLOTS
        pltpu.sync_copy(dst_hbm, dstall_v)
        pltpu.sync_copy(gate_hbm, gateall_v)

        def init(j, carry):
            sg_v[pl.ds(j * 16, 16)] = jnp.zeros((16,), jnp.float32)
            return carry

        lax.fori_loop(0, HALF_SLOTS // 16, init, 0)

        def scat(j, carry):
            idx = dstall_v[pl.ds(j * 16, 16)]
            val = gateall_v[pl.ds(j * 16, 16)]
            rel = idx - lo
            m = jnp.logical_and(rel >= 0, rel < HALF_SLOTS)
            rel = jnp.where(m, rel, 0)
            plsc.store_scatter(sg_v, [rel], val, mask=m)
            return carry

        lax.fori_loop(0, S // 16, scat, 0)
        pltpu.sync_copy(sg_v, sg_hbm.at[pl.ds(lo, HALF_SLOTS)])

    base = wid * TOK_W
    for r in range(NRND):
        pltpu.sync_copy(dst_hbm.at[pl.ds(base + r * SUB, SUB)], si_v.at[r])

    def gath(r, b, sem):
        return pltpu.async_copy(
            xs_hbm.at[pl.ds(base + r * SUB, SUB)], rows_v.at[b], sem
        )

    def scat(r, b, sem):
        return pltpu.async_copy(rows_v.at[b], disp_hbm.at[si_v.at[r]], sem)

    _run_ring(gath, scat, (gsem0, gsem1, gsem2), (ssem0, ssem1, ssem2))


@functools.partial(
    pl.kernel,
    out_type=jax.ShapeDtypeStruct((S, HIDDEN), jnp.float32),
    mesh=_MESH,
    compiler_params=pltpu.CompilerParams(needs_layout_passes=False),
    scratch_types=[
        pltpu.VMEM((NRND, SUB), jnp.int32),
        pltpu.VMEM((NRND, SUB), jnp.int32),
        pltpu.VMEM((NBUF, SUB, HIDDEN), jnp.float32),
        pltpu.SemaphoreType.DMA,
        pltpu.SemaphoreType.DMA,
        pltpu.SemaphoreType.DMA,
        pltpu.SemaphoreType.DMA,
        pltpu.SemaphoreType.DMA,
        pltpu.SemaphoreType.DMA,
    ],
)
def _combine_k(eout_hbm, dst_hbm, src_hbm, out_hbm,
               gi_v, si_v, rows_v, gsem0, gsem1, gsem2, ssem0, ssem1, ssem2):
    cid = lax.axis_index("c")
    sid = lax.axis_index("s")
    wid = sid * NC + cid
    base = wid * TOK_W
    for r in range(NRND):
        pltpu.sync_copy(dst_hbm.at[pl.ds(base + r * SUB, SUB)], gi_v.at[r])
        pltpu.sync_copy(src_hbm.at[pl.ds(base + r * SUB, SUB)], si_v.at[r])

    def gath(r, b, sem):
        return pltpu.async_copy(eout_hbm.at[gi_v.at[r]], rows_v.at[b], sem)

    def scat(r, b, sem):
        return pltpu.async_copy(rows_v.at[b], out_hbm.at[si_v.at[r]], sem)

    _run_ring(gath, scat, (gsem0, gsem1, gsem2), (ssem0, ssem1, ssem2))


def _mm_body(a_ref, w_ref, b_ref, g_ref, o_ref):
    a = a_ref[...].astype(jnp.bfloat16)
    acc = jnp.dot(a, w_ref[0], preferred_element_type=jnp.float32)
    y = acc + b_ref[pl.ds(pl.program_id(0), 1), :]
    g = g_ref[...]
    o_ref[...] = jnp.where(g > 0.0, y * g, 0.0)


def _expert_mm(disp, We, be, sg):
    nblk = CPAD // BC
    return pl.pallas_call(
        _mm_body,
        grid=(E, nblk),
        in_specs=[
            pl.BlockSpec((BC, HIDDEN), lambda e, c: (e * nblk + c, 0)),
            pl.BlockSpec((1, HIDDEN, HIDDEN), lambda e, c: (e, 0, 0)),
            pl.BlockSpec((E, HIDDEN), lambda e, c: (0, 0)),
            pl.BlockSpec((BC, 1), lambda e, c: (e * nblk + c, 0)),
        ],
        out_specs=pl.BlockSpec((BC, HIDDEN), lambda e, c: (e * nblk + c, 0)),
        out_shape=jax.ShapeDtypeStruct((SLOTS, HIDDEN), jnp.float32),
    )(disp, We, be, sg)


def kernel(features, wg, We, be):
    dst, src, gate, aux, xs = _routing(features, wg)
    dst1 = dst.reshape(S)
    src1 = src.reshape(S)
    gate1 = gate.reshape(S)

    disp, sg = _dispatch_k(xs, dst1, gate1)
    eout = _expert_mm(disp, We.astype(jnp.bfloat16), be, sg.reshape(SLOTS, 1))
    outp = _combine_k(eout, dst1, src1)

    out = outp.reshape(1, L_SEQ, N_SEQ, HIDDEN)
    return out, aux[0, 0]


# speedup vs baseline: 1.5645x; 1.2837x over previous
import functools
import math

import jax
import jax.numpy as jnp
from jax import lax
from jax.experimental import pallas as pl
from jax.experimental.pallas import tpu as pltpu
from jax.experimental.pallas import tpu_sc as plsc

HIDDEN = 1024
E = 3
L_SEQ = 2048
N_SEQ = 2
S = L_SEQ * N_SEQ
CAP = int(math.ceil(S / E))
CPAD = 1536
SLOTS = E * CPAD
HALF_SLOTS = SLOTS // 2
SGPAD = 5120

BL = 512
NBLK = L_SEQ // BL

BC = 512
DPIECE = HIDDEN // 128

NC = 2
NS = 16
NW = NC * NS
TOK_W = S // NW
SUB = 32
NRND = TOK_W // SUB
NBUF = 3


def _routing_body(feat_ref, wg_ref, dst_ref, src_ref, gate_ref, aux_ref,
                  xs_ref, acc_ref, tri_ref):
    n = pl.program_id(0)
    i = pl.program_id(1)
    first = jnp.logical_and(n == 0, i == 0)

    @pl.when(first)
    def _():
        acc_ref[...] = jnp.zeros_like(acc_ref)
        r = lax.broadcasted_iota(jnp.int32, (BL, BL), 0)
        c = lax.broadcasted_iota(jnp.int32, (BL, BL), 1)
        tri_ref[...] = (r >= c).astype(jnp.float32)

    x0 = feat_ref[0, :, 0, :]
    x1 = feat_ref[0, :, 1, :]
    x = jnp.where(n == 0, x0, x1)
    xs_ref[...] = x
    logits = jnp.dot(x, wg_ref[...], preferred_element_type=jnp.float32)

    m = jnp.max(logits, axis=1, keepdims=True)
    p = jnp.exp(logits - m)
    denom = jnp.sum(p, axis=1, keepdims=True)
    gates = p / denom

    l0 = logits[:, 0:1]
    l1 = logits[:, 1:2]
    l2 = logits[:, 2:3]
    e = jnp.where(l1 > l0, 1, 0)
    e = jnp.where(l2 > jnp.maximum(l0, l1), 2, e)

    colid = lax.broadcasted_iota(jnp.int32, (BL, E), 1)
    mask = (colid == e).astype(jnp.float32)

    prev = acc_ref[0:1, 0:E]
    cum = jnp.dot(tri_ref[...], mask, preferred_element_type=jnp.float32) + prev
    loc = jnp.sum(cum * mask, axis=1, keepdims=True) - 1.0
    loc_i = loc.astype(jnp.int32)

    kept = loc_i < CAP
    dst = e * CPAD + jnp.where(kept, loc_i, CAP)
    gate = jnp.sum(gates * mask, axis=1, keepdims=True)
    gate_eff = jnp.where(kept, gate, 0.0)

    row = lax.broadcasted_iota(jnp.int32, (BL, 1), 0)
    src = (i * BL + row) * (N_SEQ * DPIECE) + n

    dst_ref[...] = jnp.reshape(dst, (1, BL // 128, 128))
    src_ref[...] = jnp.reshape(src, (1, BL // 128, 128))
    gate_ref[...] = jnp.reshape(gate_eff, (1, BL // 128, 128))

    acc_ref[0:1, 0:E] = prev + jnp.sum(mask, axis=0, keepdims=True)
    acc_ref[1:2, 0:E] = acc_ref[1:2, 0:E] + jnp.sum(gates, axis=0, keepdims=True)

    @pl.when(jnp.logical_and(n == N_SEQ - 1, i == NBLK - 1))
    def _():
        aux = jnp.sum(acc_ref[0:1, 0:E] * acc_ref[1:2, 0:E], axis=1,
                      keepdims=True)
        aux_ref[...] = aux * (E / (S * S))


def _routing(features, wg):
    qrows = BL // 128
    return pl.pallas_call(
        _routing_body,
        grid=(N_SEQ, NBLK),
        in_specs=[
            pl.BlockSpec((1, BL, N_SEQ, HIDDEN), lambda n, i: (0, i, 0, 0)),
            pl.BlockSpec((HIDDEN, E), lambda n, i: (0, 0)),
        ],
        out_specs=[
            pl.BlockSpec((1, qrows, 128), lambda n, i: (n * NBLK + i, 0, 0)),
            pl.BlockSpec((1, qrows, 128), lambda n, i: (n * NBLK + i, 0, 0)),
            pl.BlockSpec((1, qrows, 128), lambda n, i: (n * NBLK + i, 0, 0)),
            pl.BlockSpec((1, 1), lambda n, i: (0, 0)),
            pl.BlockSpec((BL, HIDDEN), lambda n, i: (n * NBLK + i, 0)),
        ],
        out_shape=[
            jax.ShapeDtypeStruct((N_SEQ * NBLK, qrows, 128), jnp.int32),
            jax.ShapeDtypeStruct((N_SEQ * NBLK, qrows, 128), jnp.int32),
            jax.ShapeDtypeStruct((N_SEQ * NBLK, qrows, 128), jnp.float32),
            jax.ShapeDtypeStruct((1, 1), jnp.float32),
            jax.ShapeDtypeStruct((S, HIDDEN), jnp.float32),
        ],
        scratch_shapes=[
            pltpu.VMEM((8, 128), jnp.float32),
            pltpu.VMEM((BL, BL), jnp.float32),
        ],
    )(features, wg)


_MESH = plsc.VectorSubcoreMesh(
    core_axis_name="c", subcore_axis_name="s", num_cores=NC, num_subcores=NS
)


def _run_ring(issue_gather, issue_scatter, gsems, ssems):
    g = [None] * NRND
    s = [None] * NRND
    for r in range(min(NBUF, NRND)):
        g[r] = issue_gather(r, r % NBUF, gsems[r % NBUF])
    for r in range(NRND):
        b = r % NBUF
        g[r].wait()
        s[r] = issue_scatter(r, b, ssems[b])
        if r + NBUF < NRND:
            s[r].wait()
            g[r + NBUF] = issue_gather(r + NBUF, b, gsems[b])
    for r in range(max(0, NRND - NBUF), NRND):
        s[r].wait()


@functools.partial(
    pl.kernel,
    out_type=(
        jax.ShapeDtypeStruct((SLOTS, HIDDEN), jnp.float32),
        jax.ShapeDtypeStruct((SGPAD,), jnp.float32),
    ),
    mesh=_MESH,
    compiler_params=pltpu.CompilerParams(needs_layout_passes=False),
    scratch_types=[
        pltpu.VMEM((NRND, SUB), jnp.int32),
        pltpu.VMEM((NRND, SUB), jnp.int32),
        pltpu.VMEM((NBUF, SUB, HIDDEN), jnp.float32),
        pltpu.VMEM((S,), jnp.int32),
        pltpu.VMEM((S,), jnp.float32),
        pltpu.VMEM((HALF_SLOTS,), jnp.float32),
        pltpu.SemaphoreType.DMA,
        pltpu.SemaphoreType.DMA,
        pltpu.SemaphoreType.DMA,
        pltpu.SemaphoreType.DMA,
        pltpu.SemaphoreType.DMA,
        pltpu.SemaphoreType.DMA,
    ],
)
def _dispatch_k(xs_hbm, dst_hbm, gate_hbm, disp_hbm, sg_hbm,
                gi_v, si_v, rows_v, dstall_v, gateall_v, sg_v,
                gsem0, gsem1, gsem2, ssem0, ssem1, ssem2):
    cid = lax.axis_index("c")
    sid = lax.axis_index("s")
    wid = sid * NC + cid

    @pl.when(sid == 0)
    def _():
        lo = cid * HALF_SLOTS
        pltpu.sync_copy(dst_hbm, dstall_v)
        pltpu.sync_copy(gate_hbm, gateall_v)

        def init(j, carry):
            sg_v[pl.ds(j * 16, 16)] = jnp.zeros((16,), jnp.float32)
            return carry

        lax.fori_loop(0, HALF_SLOTS // 16, init, 0)

        def scat(j, carry):
            idx = dstall_v[pl.ds(j * 16, 16)]
            val = gateall_v[pl.ds(j * 16, 16)]
            rel = idx - lo
            m = jnp.logical_and(rel >= 0, rel < HALF_SLOTS)
            rel = jnp.where(m, rel, 0)
            plsc.store_scatter(sg_v, [rel], val, mask=m)
            return carry

        lax.fori_loop(0, S // 16, scat, 0)
        pltpu.sync_copy(sg_v, sg_hbm.at[pl.ds(lo, HALF_SLOTS)])

    base = wid * TOK_W
    for r in range(NRND):
        pltpu.sync_copy(dst_hbm.at[pl.ds(base + r * SUB, SUB)], si_v.at[r])

    def gath(r, b, sem):
        return pltpu.async_copy(
            xs_hbm.at[pl.ds(base + r * SUB, SUB)], rows_v.at[b], sem
        )

    def scat(r, b, sem):
        return pltpu.async_copy(rows_v.at[b], disp_hbm.at[si_v.at[r]], sem)

    _run_ring(gath, scat, (gsem0, gsem1, gsem2), (ssem0, ssem1, ssem2))


@functools.partial(
    pl.kernel,
    out_type=jax.ShapeDtypeStruct((S * DPIECE, 128), jnp.float32),
    mesh=_MESH,
    compiler_params=pltpu.CompilerParams(needs_layout_passes=False),
    scratch_types=[
        pltpu.VMEM((NRND, SUB), jnp.int32),
        pltpu.VMEM((NRND, SUB), jnp.int32),
        pltpu.VMEM((NBUF, SUB, HIDDEN), jnp.float32),
        pltpu.VMEM((NBUF, DPIECE, SUB), jnp.int32),
        pltpu.SemaphoreType.DMA,
        pltpu.SemaphoreType.DMA,
        pltpu.SemaphoreType.DMA,
        pltpu.SemaphoreType.DMA,
        pltpu.SemaphoreType.DMA,
        pltpu.SemaphoreType.DMA,
    ],
)
def _combine_k(eout_hbm, dst_hbm, src_hbm, out_hbm,
               gi_v, si_v, rows_v, iv_v,
               gsem0, gsem1, gsem2, ssem0, ssem1, ssem2):
    cid = lax.axis_index("c")
    sid = lax.axis_index("s")
    wid = sid * NC + cid
    base = wid * TOK_W
    for r in range(NRND):
        pltpu.sync_copy(dst_hbm.at[pl.ds(base + r * SUB, SUB)], gi_v.at[r])
        pltpu.sync_copy(src_hbm.at[pl.ds(base + r * SUB, SUB)], si_v.at[r])

    def gath(r, b, sem):
        return pltpu.async_copy(eout_hbm.at[gi_v.at[r]], rows_v.at[b], sem)

    def scat(r, b, sem):
        ds = []
        for k in range(DPIECE):
            for h in range(0, SUB, 16):
                iv_v[b, k, pl.ds(h, 16)] = si_v[r, pl.ds(h, 16)] + 2 * k
        for k in range(DPIECE):
            ds.append(pltpu.async_copy(
                rows_v.at[b, :, pl.ds(k * 128, 128)],
                out_hbm.at[iv_v.at[b, k]],
                sem,
            ))
        return ds

    g = [None] * NRND
    s = [None] * NRND
    gsems = (gsem0, gsem1, gsem2)
    ssems = (ssem0, ssem1, ssem2)
    for r in range(min(NBUF, NRND)):
        g[r] = gath(r, r % NBUF, gsems[r % NBUF])
    for r in range(NRND):
        b = r % NBUF
        g[r].wait()
        s[r] = scat(r, b, ssems[b])
        if r + NBUF < NRND:
            for d in s[r]:
                d.wait()
            g[r + NBUF] = gath(r + NBUF, b, gsems[b])
    for r in range(max(0, NRND - NBUF), NRND):
        for d in s[r]:
            d.wait()


def _mm_body(a_ref, w_ref, b_ref, g_ref, o_ref):
    a = a_ref[...].astype(jnp.bfloat16)
    acc = jnp.dot(a, w_ref[0], preferred_element_type=jnp.float32)
    y = acc + b_ref[pl.ds(pl.program_id(0), 1), :]
    g = g_ref[...]
    o_ref[...] = jnp.where(g > 0.0, y * g, 0.0)


def _expert_mm(disp, We, be, sg3):
    nblk = CPAD // BC
    return pl.pallas_call(
        _mm_body,
        grid=(E, nblk),
        in_specs=[
            pl.BlockSpec((BC, HIDDEN), lambda e, c: (e * nblk + c, 0)),
            pl.BlockSpec((1, HIDDEN, HIDDEN), lambda e, c: (e, 0, 0)),
            pl.BlockSpec((E, HIDDEN), lambda e, c: (0, 0)),
            pl.BlockSpec((BC, 1), lambda e, c: (e * nblk + c, 0)),
        ],
        out_specs=pl.BlockSpec((BC, HIDDEN), lambda e, c: (e * nblk + c, 0)),
        out_shape=jax.ShapeDtypeStruct((SLOTS, HIDDEN), jnp.float32),
    )(disp, We, be, sg3)


def kernel(features, wg, We, be):
    dst, src, gate, aux, xs = _routing(features, wg)
    dst1 = dst.reshape(S)
    src1 = src.reshape(S)
    gate1 = gate.reshape(S)

    disp, sg = _dispatch_k(xs, dst1, gate1)
    eout = _expert_mm(disp, We.astype(jnp.bfloat16), be, sg.reshape(SGPAD, 1))
    outp = _combine_k(eout, dst1, src1)

    out = (outp.reshape(L_SEQ, DPIECE, N_SEQ, 128)
           .transpose(0, 2, 1, 3)
           .reshape(1, L_SEQ, N_SEQ, HIDDEN))
    return out, aux[0, 0]


# speedup vs baseline: 1.5822x; 1.0113x over previous
import functools
import math

import jax
import jax.numpy as jnp
from jax import lax
from jax.experimental import pallas as pl
from jax.experimental.pallas import tpu as pltpu
from jax.experimental.pallas import tpu_sc as plsc

HIDDEN = 1024
E = 3
L_SEQ = 2048
N_SEQ = 2
S = L_SEQ * N_SEQ
CAP = int(math.ceil(S / E))
CPAD = 1536
SLOTS = E * CPAD
HALF_SLOTS = SLOTS // 2
HPACK = HIDDEN // 2

BL = 512
NBLK = L_SEQ // BL

BC = 512
DPIECE = HIDDEN // 128

NC = 2
NS = 16
NW = NC * NS
TOK_W = S // NW
SUB = 32
NRND = TOK_W // SUB
NBUF = 3


def _routing_body(feat_ref, wg_ref, dst_ref, src_ref, gate_ref, aux_ref,
                  xs_ref, acc_ref, tri_ref):
    n = pl.program_id(0)
    i = pl.program_id(1)
    first = jnp.logical_and(n == 0, i == 0)

    @pl.when(first)
    def _():
        acc_ref[...] = jnp.zeros_like(acc_ref)
        r = lax.broadcasted_iota(jnp.int32, (BL, BL), 0)
        c = lax.broadcasted_iota(jnp.int32, (BL, BL), 1)
        tri_ref[...] = (r >= c).astype(jnp.float32)

    x0 = feat_ref[0, :, 0, :]
    x1 = feat_ref[0, :, 1, :]
    x = jnp.where(n == 0, x0, x1)
    xs_ref[...] = x
    logits = jnp.dot(x, wg_ref[...], preferred_element_type=jnp.float32)

    m = jnp.max(logits, axis=1, keepdims=True)
    p = jnp.exp(logits - m)
    denom = jnp.sum(p, axis=1, keepdims=True)
    gates = p / denom

    l0 = logits[:, 0:1]
    l1 = logits[:, 1:2]
    l2 = logits[:, 2:3]
    e = jnp.where(l1 > l0, 1, 0)
    e = jnp.where(l2 > jnp.maximum(l0, l1), 2, e)

    colid = lax.broadcasted_iota(jnp.int32, (BL, E), 1)
    mask = (colid == e).astype(jnp.float32)

    prev = acc_ref[0:1, 0:E]
    cum = jnp.dot(tri_ref[...], mask, preferred_element_type=jnp.float32) + prev
    loc = jnp.sum(cum * mask, axis=1, keepdims=True) - 1.0
    loc_i = loc.astype(jnp.int32)

    kept = loc_i < CAP
    dst = e * CPAD + jnp.where(kept, loc_i, CAP)
    gate = jnp.sum(gates * mask, axis=1, keepdims=True)
    gate_eff = jnp.where(kept, gate, 0.0)

    row = lax.broadcasted_iota(jnp.int32, (BL, 1), 0)
    src = (i * BL + row) * (N_SEQ * DPIECE) + n

    dst_ref[...] = jnp.reshape(dst, (1, BL // 128, 128))
    src_ref[...] = jnp.reshape(src, (1, BL // 128, 128))
    gate_ref[...] = jnp.reshape(gate_eff, (1, BL // 128, 128))

    acc_ref[0:1, 0:E] = prev + jnp.sum(mask, axis=0, keepdims=True)
    acc_ref[1:2, 0:E] = acc_ref[1:2, 0:E] + jnp.sum(gates, axis=0, keepdims=True)

    @pl.when(jnp.logical_and(n == N_SEQ - 1, i == NBLK - 1))
    def _():
        aux = jnp.sum(acc_ref[0:1, 0:E] * acc_ref[1:2, 0:E], axis=1,
                      keepdims=True)
        aux_ref[...] = aux * (E / (S * S))


def _routing(features, wg):
    qrows = BL // 128
    return pl.pallas_call(
        _routing_body,
        grid=(N_SEQ, NBLK),
        in_specs=[
            pl.BlockSpec((1, BL, N_SEQ, HIDDEN), lambda n, i: (0, i, 0, 0)),
            pl.BlockSpec((HIDDEN, E), lambda n, i: (0, 0)),
        ],
        out_specs=[
            pl.BlockSpec((1, qrows, 128), lambda n, i: (n * NBLK + i, 0, 0)),
            pl.BlockSpec((1, qrows, 128), lambda n, i: (n * NBLK + i, 0, 0)),
            pl.BlockSpec((1, qrows, 128), lambda n, i: (n * NBLK + i, 0, 0)),
            pl.BlockSpec((1, 1), lambda n, i: (0, 0)),
            pl.BlockSpec((BL, HIDDEN), lambda n, i: (n * NBLK + i, 0)),
        ],
        out_shape=[
            jax.ShapeDtypeStruct((N_SEQ * NBLK, qrows, 128), jnp.int32),
            jax.ShapeDtypeStruct((N_SEQ * NBLK, qrows, 128), jnp.int32),
            jax.ShapeDtypeStruct((N_SEQ * NBLK, qrows, 128), jnp.float32),
            jax.ShapeDtypeStruct((1, 1), jnp.float32),
            jax.ShapeDtypeStruct((S, HIDDEN), jnp.float32),
        ],
        scratch_shapes=[
            pltpu.VMEM((8, 128), jnp.float32),
            pltpu.VMEM((BL, BL), jnp.float32),
        ],
    )(features, wg)


_MESH = plsc.VectorSubcoreMesh(
    core_axis_name="c", subcore_axis_name="s", num_cores=NC, num_subcores=NS
)


def _run_ring(issue_gather, issue_scatter, gsems, ssems):
    g = [None] * NRND
    s = [None] * NRND
    for r in range(min(NBUF, NRND)):
        g[r] = issue_gather(r, r % NBUF, gsems[r % NBUF])
    for r in range(NRND):
        b = r % NBUF
        g[r].wait()
        s[r] = issue_scatter(r, b, ssems[b])
        if r + NBUF < NRND:
            s[r].wait()
            g[r + NBUF] = issue_gather(r + NBUF, b, gsems[b])
    for r in range(max(0, NRND - NBUF), NRND):
        s[r].wait()


@functools.partial(
    pl.kernel,
    out_type=(
        jax.ShapeDtypeStruct((SLOTS, HIDDEN), jnp.float32),
        jax.ShapeDtypeStruct((SLOTS,), jnp.float32),
    ),
    mesh=_MESH,
    compiler_params=pltpu.CompilerParams(needs_layout_passes=False),
    scratch_types=[
        pltpu.VMEM((NRND, SUB), jnp.int32),
        pltpu.VMEM((NRND, SUB), jnp.int32),
        pltpu.VMEM((NBUF, SUB, HIDDEN), jnp.float32),
        pltpu.VMEM((S,), jnp.int32),
        pltpu.VMEM((S,), jnp.float32),
        pltpu.VMEM((HALF_SLOTS,), jnp.float32),
        pltpu.SemaphoreType.DMA,
        pltpu.SemaphoreType.DMA,
        pltpu.SemaphoreType.DMA,
        pltpu.SemaphoreType.DMA,
        pltpu.SemaphoreType.DMA,
        pltpu.SemaphoreType.DMA,
    ],
)
def _dispatch_k(xs_hbm, dst_hbm, gate_hbm, disp_hbm, sg_hbm,
                gi_v, si_v, rows_v, dstall_v, gateall_v, sg_v,
                gsem0, gsem1, gsem2, ssem0, ssem1, ssem2):
    cid = lax.axis_index("c")
    sid = lax.axis_index("s")
    wid = sid * NC + cid

    @pl.when(sid == 0)
    def _():
        lo = cid * HALF_SLOTS
        pltpu.sync_copy(dst_hbm, dstall_v)
        pltpu.sync_copy(gate_hbm, gateall_v)

        def init(j, carry):
            sg_v[pl.ds(j * 16, 16)] = jnp.zeros((16,), jnp.float32)
            return carry

        lax.fori_loop(0, HALF_SLOTS // 16, init, 0)

        def scat(j, carry):
            idx = dstall_v[pl.ds(j * 16, 16)]
            val = gateall_v[pl.ds(j * 16, 16)]
            rel = idx - lo
            m = jnp.logical_and(rel >= 0, rel < HALF_SLOTS)
            rel = jnp.where(m, rel, 0)
            plsc.store_scatter(sg_v, [rel], val, mask=m)
            return carry

        lax.fori_loop(0, S // 16, scat, 0)
        pltpu.sync_copy(sg_v, sg_hbm.at[pl.ds(lo, HALF_SLOTS)])

    base = wid * TOK_W
    for r in range(NRND):
        pltpu.sync_copy(dst_hbm.at[pl.ds(base + r * SUB, SUB)], si_v.at[r])

    def gath(r, b, sem):
        return pltpu.async_copy(
            xs_hbm.at[pl.ds(base + r * SUB, SUB)], rows_v.at[b], sem
        )

    def scat(r, b, sem):
        return pltpu.async_copy(rows_v.at[b], disp_hbm.at[si_v.at[r]], sem)

    _run_ring(gath, scat, (gsem0, gsem1, gsem2), (ssem0, ssem1, ssem2))


@functools.partial(
    pl.kernel,
    out_type=jax.ShapeDtypeStruct((S * DPIECE, 128), jnp.float32),
    mesh=_MESH,
    compiler_params=pltpu.CompilerParams(needs_layout_passes=False),
    scratch_types=[
        pltpu.VMEM((NRND, SUB), jnp.int32),
        pltpu.VMEM((NRND, SUB), jnp.int32),
        pltpu.VMEM((NBUF, SUB, HIDDEN), jnp.float32),
        pltpu.VMEM((NBUF, DPIECE, SUB), jnp.int32),
        pltpu.SemaphoreType.DMA,
        pltpu.SemaphoreType.DMA,
        pltpu.SemaphoreType.DMA,
        pltpu.SemaphoreType.DMA,
        pltpu.SemaphoreType.DMA,
        pltpu.SemaphoreType.DMA,
    ],
)
def _combine_k(eout_hbm, dst_hbm, src_hbm, out_hbm,
               gi_v, si_v, rows_v, iv_v,
               gsem0, gsem1, gsem2, ssem0, ssem1, ssem2):
    cid = lax.axis_index("c")
    sid = lax.axis_index("s")
    wid = sid * NC + cid
    base = wid * TOK_W
    for r in range(NRND):
        pltpu.sync_copy(dst_hbm.at[pl.ds(base + r * SUB, SUB)], gi_v.at[r])
        pltpu.sync_copy(src_hbm.at[pl.ds(base + r * SUB, SUB)], si_v.at[r])

    def gath(r, b, sem):
        return pltpu.async_copy(eout_hbm.at[gi_v.at[r]], rows_v.at[b], sem)

    def scat(r, b, sem):
        ds = []
        for k in range(DPIECE):
            for h in range(0, SUB, 16):
                iv_v[b, k, pl.ds(h, 16)] = si_v[r, pl.ds(h, 16)] + 2 * k
        for k in range(DPIECE):
            ds.append(pltpu.async_copy(
                rows_v.at[b, :, pl.ds(k * 128, 128)],
                out_hbm.at[iv_v.at[b, k]],
                sem,
            ))
        return ds

    g = [None] * NRND
    s = [None] * NRND
    gsems = (gsem0, gsem1, gsem2)
    ssems = (ssem0, ssem1, ssem2)
    for r in range(min(NBUF, NRND)):
        g[r] = gath(r, r % NBUF, gsems[r % NBUF])
    for r in range(NRND):
        b = r % NBUF
        g[r].wait()
        s[r] = scat(r, b, ssems[b])
        if r + NBUF < NRND:
            for d in s[r]:
                d.wait()
            g[r + NBUF] = gath(r + NBUF, b, gsems[b])
    for r in range(max(0, NRND - NBUF), NRND):
        for d in s[r]:
            d.wait()


def _mm_body(a_ref, w_ref, b_ref, g_ref, o_ref):
    a = a_ref[...].astype(jnp.bfloat16)
    acc = jnp.dot(a, w_ref[0], preferred_element_type=jnp.float32)
    y = acc + b_ref[pl.ds(pl.program_id(0), 1), :]
    g = jnp.transpose(g_ref[...])
    o_ref[...] = jnp.where(g > 0.0, y * g, 0.0)


def _expert_mm(disp, We, be, sg):
    nblk = CPAD // BC
    return pl.pallas_call(
        _mm_body,
        grid=(E, nblk),
        in_specs=[
            pl.BlockSpec((BC, HIDDEN), lambda e, c: (e * nblk + c, 0)),
            pl.BlockSpec((1, HIDDEN, HIDDEN), lambda e, c: (e, 0, 0)),
            pl.BlockSpec((E, HIDDEN), lambda e, c: (0, 0)),
            pl.BlockSpec((1, BC), lambda e, c: (0, e * nblk + c)),
        ],
        out_specs=pl.BlockSpec((BC, HIDDEN), lambda e, c: (e * nblk + c, 0)),
        out_shape=jax.ShapeDtypeStruct((SLOTS, HIDDEN), jnp.float32),
    )(disp, We, be, sg)


def kernel(features, wg, We, be):
    dst, src, gate, aux, xs = _routing(features, wg)
    dst1 = dst.reshape(S)
    src1 = src.reshape(S)
    gate1 = gate.reshape(S)

    disp, sg = _dispatch_k(xs, dst1, gate1)
    eout = _expert_mm(disp, We.astype(jnp.bfloat16), be, sg.reshape(1, SLOTS))
    outp = _combine_k(eout, dst1, src1)

    out = (outp.reshape(L_SEQ, DPIECE, N_SEQ, 128)
           .transpose(0, 2, 1, 3)
           .reshape(1, L_SEQ, N_SEQ, HIDDEN))
    return out, aux[0, 0]


# speedup vs baseline: 1.5919x; 1.0061x over previous
import functools
import math

import jax
import jax.numpy as jnp
from jax import lax
from jax.experimental import pallas as pl
from jax.experimental.pallas import tpu as pltpu
from jax.experimental.pallas import tpu_sc as plsc

HIDDEN = 1024
E = 3
L_SEQ = 2048
N_SEQ = 2
S = L_SEQ * N_SEQ
CAP = int(math.ceil(S / E))
CPAD = 1536
SLOTS = E * CPAD
HALF_SLOTS = SLOTS // 2
HPACK = HIDDEN // 2

BL = 1024
NBLK = L_SEQ // BL

BC = 512
DPIECE = HIDDEN // 128

NC = 2
NS = 16
NW = NC * NS
TOK_W = S // NW
SUB = 32
NRND = TOK_W // SUB
NBUF = 3


def _routing_body(feat_ref, wg_ref, dst_ref, src_ref, gate_ref, aux_ref,
                  xs_ref, acc_ref, tri_ref):
    n = pl.program_id(0)
    i = pl.program_id(1)
    first = jnp.logical_and(n == 0, i == 0)

    @pl.when(first)
    def _():
        acc_ref[...] = jnp.zeros_like(acc_ref)
        r = lax.broadcasted_iota(jnp.int32, (BL, BL), 0)
        c = lax.broadcasted_iota(jnp.int32, (BL, BL), 1)
        tri_ref[...] = (r >= c).astype(jnp.float32)

    x0 = feat_ref[0, :, 0, :]
    x1 = feat_ref[0, :, 1, :]
    x = jnp.where(n == 0, x0, x1)
    xs_ref[...] = x
    logits = lax.dot_general(
        x, wg_ref[...], (((1,), (1,)), ((), ())),
        preferred_element_type=jnp.float32)

    m = jnp.max(logits, axis=1, keepdims=True)
    p = jnp.exp(logits - m)
    denom = jnp.sum(p, axis=1, keepdims=True)
    gates = p / denom

    l0 = logits[:, 0:1]
    l1 = logits[:, 1:2]
    l2 = logits[:, 2:3]
    e = jnp.where(l1 > l0, 1, 0)
    e = jnp.where(l2 > jnp.maximum(l0, l1), 2, e)

    colid = lax.broadcasted_iota(jnp.int32, (BL, E), 1)
    mask = (colid == e).astype(jnp.float32)

    prev = acc_ref[0:1, 0:E]
    cum = jnp.dot(tri_ref[...], mask, preferred_element_type=jnp.float32) + prev
    loc = jnp.sum(cum * mask, axis=1, keepdims=True) - 1.0
    loc_i = loc.astype(jnp.int32)

    kept = loc_i < CAP
    dst = e * CPAD + jnp.where(kept, loc_i, CAP)
    gate = jnp.sum(gates * mask, axis=1, keepdims=True)
    gate_eff = jnp.where(kept, gate, 0.0)

    row = lax.broadcasted_iota(jnp.int32, (BL, 1), 0)
    src = (i * BL + row) * (N_SEQ * DPIECE) + n

    dst_ref[...] = jnp.reshape(dst, (1, BL // 128, 128))
    src_ref[...] = jnp.reshape(src, (1, BL // 128, 128))
    gate_ref[...] = jnp.reshape(gate_eff, (1, BL // 128, 128))

    acc_ref[0:1, 0:E] = prev + jnp.sum(mask, axis=0, keepdims=True)
    acc_ref[1:2, 0:E] = acc_ref[1:2, 0:E] + jnp.sum(gates, axis=0, keepdims=True)

    @pl.when(jnp.logical_and(n == N_SEQ - 1, i == NBLK - 1))
    def _():
        aux = jnp.sum(acc_ref[0:1, 0:E] * acc_ref[1:2, 0:E], axis=1,
                      keepdims=True)
        aux_ref[...] = aux * (E / (S * S))


def _routing(features, wg):
    qrows = BL // 128
    return pl.pallas_call(
        _routing_body,
        grid=(N_SEQ, NBLK),
        in_specs=[
            pl.BlockSpec((1, BL, N_SEQ, HIDDEN), lambda n, i: (0, i, 0, 0)),
            pl.BlockSpec((E, HIDDEN), lambda n, i: (0, 0)),
        ],
        out_specs=[
            pl.BlockSpec((1, qrows, 128), lambda n, i: (n * NBLK + i, 0, 0)),
            pl.BlockSpec((1, qrows, 128), lambda n, i: (n * NBLK + i, 0, 0)),
            pl.BlockSpec((1, qrows, 128), lambda n, i: (n * NBLK + i, 0, 0)),
            pl.BlockSpec((1, 1), lambda n, i: (0, 0)),
            pl.BlockSpec((BL, HIDDEN), lambda n, i: (n * NBLK + i, 0)),
        ],
        out_shape=[
            jax.ShapeDtypeStruct((N_SEQ * NBLK, qrows, 128), jnp.int32),
            jax.ShapeDtypeStruct((N_SEQ * NBLK, qrows, 128), jnp.int32),
            jax.ShapeDtypeStruct((N_SEQ * NBLK, qrows, 128), jnp.float32),
            jax.ShapeDtypeStruct((1, 1), jnp.float32),
            jax.ShapeDtypeStruct((S, HIDDEN), jnp.float32),
        ],
        scratch_shapes=[
            pltpu.VMEM((8, 128), jnp.float32),
            pltpu.VMEM((BL, BL), jnp.float32),
        ],
    )(features, wg)


_MESH = plsc.VectorSubcoreMesh(
    core_axis_name="c", subcore_axis_name="s", num_cores=NC, num_subcores=NS
)


def _run_ring(issue_gather, issue_scatter, gsems, ssems):
    g = [None] * NRND
    s = [None] * NRND
    for r in range(min(NBUF, NRND)):
        g[r] = issue_gather(r, r % NBUF, gsems[r % NBUF])
    for r in range(NRND):
        b = r % NBUF
        g[r].wait()
        s[r] = issue_scatter(r, b, ssems[b])
        if r + NBUF < NRND:
            s[r].wait()
            g[r + NBUF] = issue_gather(r + NBUF, b, gsems[b])
    for r in range(max(0, NRND - NBUF), NRND):
        s[r].wait()


@functools.partial(
    pl.kernel,
    out_type=(
        jax.ShapeDtypeStruct((SLOTS, HIDDEN), jnp.float32),
        jax.ShapeDtypeStruct((SLOTS,), jnp.float32),
    ),
    mesh=_MESH,
    compiler_params=pltpu.CompilerParams(needs_layout_passes=False),
    scratch_types=[
        pltpu.VMEM((NRND, SUB), jnp.int32),
        pltpu.VMEM((NRND, SUB), jnp.int32),
        pltpu.VMEM((NBUF, SUB, HIDDEN), jnp.float32),
        pltpu.VMEM((S,), jnp.int32),
        pltpu.VMEM((S,), jnp.float32),
        pltpu.VMEM((HALF_SLOTS,), jnp.float32),
        pltpu.SemaphoreType.DMA,
        pltpu.SemaphoreType.DMA,
        pltpu.SemaphoreType.DMA,
        pltpu.SemaphoreType.DMA,
        pltpu.SemaphoreType.DMA,
        pltpu.SemaphoreType.DMA,
    ],
)
def _dispatch_k(xs_hbm, dst_hbm, gate_hbm, disp_hbm, sg_hbm,
                gi_v, si_v, rows_v, dstall_v, gateall_v, sg_v,
                gsem0, gsem1, gsem2, ssem0, ssem1, ssem2):
    cid = lax.axis_index("c")
    sid = lax.axis_index("s")
    wid = sid * NC + cid

    @pl.when(sid == 0)
    def _():
        lo = cid * HALF_SLOTS
        pltpu.sync_copy(dst_hbm, dstall_v)
        pltpu.sync_copy(gate_hbm, gateall_v)

        def init(j, carry):
            sg_v[pl.ds(j * 16, 16)] = jnp.zeros((16,), jnp.float32)
            return carry

        lax.fori_loop(0, HALF_SLOTS // 16, init, 0)

        def scat(j, carry):
            idx = dstall_v[pl.ds(j * 16, 16)]
            val = gateall_v[pl.ds(j * 16, 16)]
            rel = idx - lo
            m = jnp.logical_and(rel >= 0, rel < HALF_SLOTS)
            rel = jnp.where(m, rel, 0)
            plsc.store_scatter(sg_v, [rel], val, mask=m)
            return carry

        lax.fori_loop(0, S // 16, scat, 0)
        pltpu.sync_copy(sg_v, sg_hbm.at[pl.ds(lo, HALF_SLOTS)])

    base = wid * TOK_W
    for r in range(NRND):
        pltpu.sync_copy(dst_hbm.at[pl.ds(base + r * SUB, SUB)], si_v.at[r])

    def gath(r, b, sem):
        return pltpu.async_copy(
            xs_hbm.at[pl.ds(base + r * SUB, SUB)], rows_v.at[b], sem
        )

    def scat(r, b, sem):
        return pltpu.async_copy(rows_v.at[b], disp_hbm.at[si_v.at[r]], sem)

    _run_ring(gath, scat, (gsem0, gsem1, gsem2), (ssem0, ssem1, ssem2))


@functools.partial(
    pl.kernel,
    out_type=jax.ShapeDtypeStruct((S * DPIECE, 128), jnp.float32),
    mesh=_MESH,
    compiler_params=pltpu.CompilerParams(needs_layout_passes=False),
    scratch_types=[
        pltpu.VMEM((NRND, SUB), jnp.int32),
        pltpu.VMEM((NRND, SUB), jnp.int32),
        pltpu.VMEM((NBUF, SUB, HIDDEN), jnp.float32),
        pltpu.VMEM((NBUF, DPIECE, SUB), jnp.int32),
        pltpu.SemaphoreType.DMA,
        pltpu.SemaphoreType.DMA,
        pltpu.SemaphoreType.DMA,
        pltpu.SemaphoreType.DMA,
        pltpu.SemaphoreType.DMA,
        pltpu.SemaphoreType.DMA,
    ],
)
def _combine_k(eout_hbm, dst_hbm, src_hbm, out_hbm,
               gi_v, si_v, rows_v, iv_v,
               gsem0, gsem1, gsem2, ssem0, ssem1, ssem2):
    cid = lax.axis_index("c")
    sid = lax.axis_index("s")
    wid = sid * NC + cid
    base = wid * TOK_W
    for r in range(NRND):
        pltpu.sync_copy(dst_hbm.at[pl.ds(base + r * SUB, SUB)], gi_v.at[r])
        pltpu.sync_copy(src_hbm.at[pl.ds(base + r * SUB, SUB)], si_v.at[r])

    def gath(r, b, sem):
        return pltpu.async_copy(eout_hbm.at[gi_v.at[r]], rows_v.at[b], sem)

    def scat(r, b, sem):
        ds = []
        for k in range(DPIECE):
            for h in range(0, SUB, 16):
                iv_v[b, k, pl.ds(h, 16)] = si_v[r, pl.ds(h, 16)] + 2 * k
        for k in range(DPIECE):
            ds.append(pltpu.async_copy(
                rows_v.at[b, :, pl.ds(k * 128, 128)],
                out_hbm.at[iv_v.at[b, k]],
                sem,
            ))
        return ds

    g = [None] * NRND
    s = [None] * NRND
    gsems = (gsem0, gsem1, gsem2)
    ssems = (ssem0, ssem1, ssem2)
    for r in range(min(NBUF, NRND)):
        g[r] = gath(r, r % NBUF, gsems[r % NBUF])
    for r in range(NRND):
        b = r % NBUF
        g[r].wait()
        s[r] = scat(r, b, ssems[b])
        if r + NBUF < NRND:
            for d in s[r]:
                d.wait()
            g[r + NBUF] = gath(r + NBUF, b, gsems[b])
    for r in range(max(0, NRND - NBUF), NRND):
        for d in s[r]:
            d.wait()


def _mm_body(a_ref, w_ref, b_ref, g_ref, o_ref):
    a = a_ref[...].astype(jnp.bfloat16)
    acc = jnp.dot(a, w_ref[0], preferred_element_type=jnp.float32)
    y = acc + b_ref[pl.ds(pl.program_id(0), 1), :]
    g = jnp.transpose(g_ref[...])
    o_ref[...] = jnp.where(g > 0.0, y * g, 0.0)


def _expert_mm(disp, We, be, sg):
    nblk = CPAD // BC
    return pl.pallas_call(
        _mm_body,
        grid=(E, nblk),
        in_specs=[
            pl.BlockSpec((BC, HIDDEN), lambda e, c: (e * nblk + c, 0)),
            pl.BlockSpec((1, HIDDEN, HIDDEN), lambda e, c: (e, 0, 0)),
            pl.BlockSpec((E, HIDDEN), lambda e, c: (0, 0)),
            pl.BlockSpec((1, BC), lambda e, c: (0, e * nblk + c)),
        ],
        out_specs=pl.BlockSpec((BC, HIDDEN), lambda e, c: (e * nblk + c, 0)),
        out_shape=jax.ShapeDtypeStruct((SLOTS, HIDDEN), jnp.float32),
    )(disp, We, be, sg)


def kernel(features, wg, We, be):
    dst, src, gate, aux, xs = _routing(features, wg.T)
    dst1 = dst.reshape(S)
    src1 = src.reshape(S)
    gate1 = gate.reshape(S)

    disp, sg = _dispatch_k(xs, dst1, gate1)
    eout = _expert_mm(disp, We.astype(jnp.bfloat16), be, sg.reshape(1, SLOTS))
    outp = _combine_k(eout, dst1, src1)

    out = (outp.reshape(L_SEQ, DPIECE, N_SEQ, 128)
           .transpose(0, 2, 1, 3)
           .reshape(1, L_SEQ, N_SEQ, HIDDEN))
    return out, aux[0, 0]


# speedup vs baseline: 1.6094x; 1.0110x over previous
import functools
import math

import jax
import jax.numpy as jnp
from jax import lax
from jax.experimental import pallas as pl
from jax.experimental.pallas import tpu as pltpu
from jax.experimental.pallas import tpu_sc as plsc

HIDDEN = 1024
E = 3
L_SEQ = 2048
N_SEQ = 2
S = L_SEQ * N_SEQ
CAP = int(math.ceil(S / E))
CPAD = 1536
SLOTS = E * CPAD
HALF_SLOTS = SLOTS // 2
HPACK = HIDDEN // 2

BL = 512
NBLK = L_SEQ // BL

BC = 512
DPIECE = HIDDEN // 128

NC = 2
NS = 16
NW = NC * NS
TOK_W = S // NW
SUB = 32
NRND = TOK_W // SUB
NBUF = 3


def _routing_body(feat_ref, wg_ref, dst_ref, src_ref, gate_ref, aux_ref,
                  xs_ref, acc_ref, tri_ref):
    n = pl.program_id(0)
    i = pl.program_id(1)
    first = jnp.logical_and(n == 0, i == 0)

    @pl.when(first)
    def _():
        acc_ref[...] = jnp.zeros_like(acc_ref)
        r = lax.broadcasted_iota(jnp.int32, (BL, BL), 0)
        c = lax.broadcasted_iota(jnp.int32, (BL, BL), 1)
        tri_ref[...] = (r >= c).astype(jnp.float32)

    x0 = feat_ref[0, :, 0, :]
    x1 = feat_ref[0, :, 1, :]
    x = jnp.where(n == 0, x0, x1)
    xs_ref[...] = x
    logits = lax.dot_general(
        x, wg_ref[...], (((1,), (1,)), ((), ())),
        preferred_element_type=jnp.float32)

    m = jnp.max(logits, axis=1, keepdims=True)
    p = jnp.exp(logits - m)
    denom = jnp.sum(p, axis=1, keepdims=True)
    gates = p / denom

    l0 = logits[:, 0:1]
    l1 = logits[:, 1:2]
    l2 = logits[:, 2:3]
    e = jnp.where(l1 > l0, 1, 0)
    e = jnp.where(l2 > jnp.maximum(l0, l1), 2, e)

    colid = lax.broadcasted_iota(jnp.int32, (BL, E), 1)
    mask = (colid == e).astype(jnp.float32)

    prev = acc_ref[0:1, 0:E]
    cum = jnp.dot(tri_ref[...], mask, preferred_element_type=jnp.float32) + prev
    loc = jnp.sum(cum * mask, axis=1, keepdims=True) - 1.0
    loc_i = loc.astype(jnp.int32)

    kept = loc_i < CAP
    dst = e * CPAD + jnp.where(kept, loc_i, CAP)
    gate = jnp.sum(gates * mask, axis=1, keepdims=True)
    gate_eff = jnp.where(kept, gate, 0.0)

    row = lax.broadcasted_iota(jnp.int32, (BL, 1), 0)
    src = (i * BL + row) * (N_SEQ * DPIECE) + n

    dst_ref[...] = jnp.reshape(dst, (1, BL // 128, 128))
    src_ref[...] = jnp.reshape(src, (1, BL // 128, 128))
    gate_ref[...] = jnp.reshape(gate_eff, (1, BL // 128, 128))

    acc_ref[0:1, 0:E] = prev + jnp.sum(mask, axis=0, keepdims=True)
    acc_ref[1:2, 0:E] = acc_ref[1:2, 0:E] + jnp.sum(gates, axis=0, keepdims=True)

    @pl.when(jnp.logical_and(n == N_SEQ - 1, i == NBLK - 1))
    def _():
        aux = jnp.sum(acc_ref[0:1, 0:E] * acc_ref[1:2, 0:E], axis=1,
                      keepdims=True)
        aux_ref[...] = aux * (E / (S * S))


def _routing(features, wg):
    qrows = BL // 128
    return pl.pallas_call(
        _routing_body,
        grid=(N_SEQ, NBLK),
        in_specs=[
            pl.BlockSpec((1, BL, N_SEQ, HIDDEN), lambda n, i: (0, i, 0, 0)),
            pl.BlockSpec((E, HIDDEN), lambda n, i: (0, 0)),
        ],
        out_specs=[
            pl.BlockSpec((1, qrows, 128), lambda n, i: (n * NBLK + i, 0, 0)),
            pl.BlockSpec((1, qrows, 128), lambda n, i: (n * NBLK + i, 0, 0)),
            pl.BlockSpec((1, qrows, 128), lambda n, i: (n * NBLK + i, 0, 0)),
            pl.BlockSpec((1, 1), lambda n, i: (0, 0)),
            pl.BlockSpec((BL, HIDDEN), lambda n, i: (n * NBLK + i, 0)),
        ],
        out_shape=[
            jax.ShapeDtypeStruct((N_SEQ * NBLK, qrows, 128), jnp.int32),
            jax.ShapeDtypeStruct((N_SEQ * NBLK, qrows, 128), jnp.int32),
            jax.ShapeDtypeStruct((N_SEQ * NBLK, qrows, 128), jnp.float32),
            jax.ShapeDtypeStruct((1, 1), jnp.float32),
            jax.ShapeDtypeStruct((S, HIDDEN), jnp.float32),
        ],
        scratch_shapes=[
            pltpu.VMEM((8, 128), jnp.float32),
            pltpu.VMEM((BL, BL), jnp.float32),
        ],
    )(features, wg)


_MESH = plsc.VectorSubcoreMesh(
    core_axis_name="c", subcore_axis_name="s", num_cores=NC, num_subcores=NS
)


def _run_ring(issue_gather, issue_scatter, gsems, ssems):
    g = [None] * NRND
    s = [None] * NRND
    for r in range(min(NBUF, NRND)):
        g[r] = issue_gather(r, r % NBUF, gsems[r % NBUF])
    for r in range(NRND):
        b = r % NBUF
        g[r].wait()
        s[r] = issue_scatter(r, b, ssems[b])
        if r + NBUF < NRND:
            s[r].wait()
            g[r + NBUF] = issue_gather(r + NBUF, b, gsems[b])
    for r in range(max(0, NRND - NBUF), NRND):
        s[r].wait()


@functools.partial(
    pl.kernel,
    out_type=(
        jax.ShapeDtypeStruct((SLOTS, HIDDEN), jnp.float32),
        jax.ShapeDtypeStruct((SLOTS,), jnp.float32),
    ),
    mesh=_MESH,
    compiler_params=pltpu.CompilerParams(needs_layout_passes=False),
    scratch_types=[
        pltpu.VMEM((NRND, SUB), jnp.int32),
        pltpu.VMEM((NRND, SUB), jnp.int32),
        pltpu.VMEM((NBUF, SUB, HIDDEN), jnp.float32),
        pltpu.VMEM((S,), jnp.int32),
        pltpu.VMEM((S,), jnp.float32),
        pltpu.VMEM((HALF_SLOTS,), jnp.float32),
        pltpu.SemaphoreType.DMA,
        pltpu.SemaphoreType.DMA,
        pltpu.SemaphoreType.DMA,
        pltpu.SemaphoreType.DMA,
        pltpu.SemaphoreType.DMA,
        pltpu.SemaphoreType.DMA,
    ],
)
def _dispatch_k(xs_hbm, dst_hbm, gate_hbm, disp_hbm, sg_hbm,
                gi_v, si_v, rows_v, dstall_v, gateall_v, sg_v,
                gsem0, gsem1, gsem2, ssem0, ssem1, ssem2):
    cid = lax.axis_index("c")
    sid = lax.axis_index("s")
    wid = sid * NC + cid

    @pl.when(sid == 0)
    def _():
        lo = cid * HALF_SLOTS
        pltpu.sync_copy(dst_hbm, dstall_v)
        pltpu.sync_copy(gate_hbm, gateall_v)

        def init(j, carry):
            sg_v[pl.ds(j * 16, 16)] = jnp.zeros((16,), jnp.float32)
            return carry

        lax.fori_loop(0, HALF_SLOTS // 16, init, 0)

        def scat(j, carry):
            idx = dstall_v[pl.ds(j * 16, 16)]
            val = gateall_v[pl.ds(j * 16, 16)]
            rel = idx - lo
            m = jnp.logical_and(rel >= 0, rel < HALF_SLOTS)
            rel = jnp.where(m, rel, 0)
            plsc.store_scatter(sg_v, [rel], val, mask=m)
            return carry

        lax.fori_loop(0, S // 16, scat, 0)
        pltpu.sync_copy(sg_v, sg_hbm.at[pl.ds(lo, HALF_SLOTS)])

    base = wid * TOK_W
    for r in range(NRND):
        pltpu.sync_copy(dst_hbm.at[pl.ds(base + r * SUB, SUB)], si_v.at[r])

    def gath(r, b, sem):
        return pltpu.async_copy(
            xs_hbm.at[pl.ds(base + r * SUB, SUB)], rows_v.at[b], sem
        )

    def scat(r, b, sem):
        return pltpu.async_copy(rows_v.at[b], disp_hbm.at[si_v.at[r]], sem)

    _run_ring(gath, scat, (gsem0, gsem1, gsem2), (ssem0, ssem1, ssem2))


@functools.partial(
    pl.kernel,
    out_type=jax.ShapeDtypeStruct((S * DPIECE, 128), jnp.float32),
    mesh=_MESH,
    compiler_params=pltpu.CompilerParams(needs_layout_passes=False),
    scratch_types=[
        pltpu.VMEM((NRND, SUB), jnp.int32),
        pltpu.VMEM((NRND, SUB), jnp.int32),
        pltpu.VMEM((NBUF, SUB, HIDDEN), jnp.float32),
        pltpu.VMEM((NBUF, DPIECE, SUB), jnp.int32),
        pltpu.SemaphoreType.DMA,
        pltpu.SemaphoreType.DMA,
        pltpu.SemaphoreType.DMA,
        pltpu.SemaphoreType.DMA,
        pltpu.SemaphoreType.DMA,
        pltpu.SemaphoreType.DMA,
    ],
)
def _combine_k(eout_hbm, dst_hbm, src_hbm, out_hbm,
               gi_v, si_v, rows_v, iv_v,
               gsem0, gsem1, gsem2, ssem0, ssem1, ssem2):
    cid = lax.axis_index("c")
    sid = lax.axis_index("s")
    wid = sid * NC + cid
    base = wid * TOK_W
    for r in range(NRND):
        pltpu.sync_copy(dst_hbm.at[pl.ds(base + r * SUB, SUB)], gi_v.at[r])
        pltpu.sync_copy(src_hbm.at[pl.ds(base + r * SUB, SUB)], si_v.at[r])

    def gath(r, b, sem):
        return pltpu.async_copy(eout_hbm.at[gi_v.at[r]], rows_v.at[b], sem)

    def scat(r, b, sem):
        ds = []
        for k in range(DPIECE):
            for h in range(0, SUB, 16):
                iv_v[b, k, pl.ds(h, 16)] = si_v[r, pl.ds(h, 16)] + 2 * k
        for k in range(DPIECE):
            ds.append(pltpu.async_copy(
                rows_v.at[b, :, pl.ds(k * 128, 128)],
                out_hbm.at[iv_v.at[b, k]],
                sem,
            ))
        return ds

    g = [None] * NRND
    s = [None] * NRND
    gsems = (gsem0, gsem1, gsem2)
    ssems = (ssem0, ssem1, ssem2)
    for r in range(min(NBUF, NRND)):
        g[r] = gath(r, r % NBUF, gsems[r % NBUF])
    for r in range(NRND):
        b = r % NBUF
        g[r].wait()
        s[r] = scat(r, b, ssems[b])
        if r + NBUF < NRND:
            for d in s[r]:
                d.wait()
            g[r + NBUF] = gath(r + NBUF, b, gsems[b])
    for r in range(max(0, NRND - NBUF), NRND):
        for d in s[r]:
            d.wait()


def _mm_body(a_ref, w_ref, b_ref, g_ref, o_ref):
    a = a_ref[...].astype(jnp.bfloat16)
    acc = jnp.dot(a, w_ref[0], preferred_element_type=jnp.float32)
    y = acc + b_ref[pl.ds(pl.program_id(0), 1), :]
    g = jnp.transpose(g_ref[...])
    o_ref[...] = jnp.where(g > 0.0, y * g, 0.0)


def _expert_mm(disp, We, be, sg):
    nblk = CPAD // BC
    return pl.pallas_call(
        _mm_body,
        grid=(E, nblk),
        in_specs=[
            pl.BlockSpec((BC, HIDDEN), lambda e, c: (e * nblk + c, 0)),
            pl.BlockSpec((1, HIDDEN, HIDDEN), lambda e, c: (e, 0, 0)),
            pl.BlockSpec((E, HIDDEN), lambda e, c: (0, 0)),
            pl.BlockSpec((1, BC), lambda e, c: (0, e * nblk + c)),
        ],
        out_specs=pl.BlockSpec((BC, HIDDEN), lambda e, c: (e * nblk + c, 0)),
        out_shape=jax.ShapeDtypeStruct((SLOTS, HIDDEN), jnp.float32),
    )(disp, We, be, sg)


def kernel(features, wg, We, be):
    dst, src, gate, aux, xs = _routing(features, wg.T)
    dst1 = dst.reshape(S)
    src1 = src.reshape(S)
    gate1 = gate.reshape(S)

    disp, sg = _dispatch_k(xs, dst1, gate1)
    eout = _expert_mm(disp, We.astype(jnp.bfloat16), be, sg.reshape(1, SLOTS))
    outp = _combine_k(eout, dst1, src1)

    out = (outp.reshape(L_SEQ, DPIECE, N_SEQ, 128)
           .transpose(0, 2, 1, 3)
           .reshape(1, L_SEQ, N_SEQ, HIDDEN))
    return out, aux[0, 0]
